# Initial kernel scaffold; baseline (speedup 1.0000x reference)
import jax
import jax.numpy as jnp
from jax.experimental import pallas as pl


def kernel(raw_features, wl_role_ids, init_pos_ids, hop_dis_ids, W, b, wl_table, pos_table, hop_table, gamma, beta):
    raise NotImplementedError("write your pallas kernel here")


# speedup vs baseline: 3.5417x; 3.5417x over previous
import functools

import jax
import jax.numpy as jnp
from jax import lax
from jax.experimental import pallas as pl
from jax.experimental.pallas import tpu as pltpu
from jax.experimental.pallas import tpu_sc as plsc

X_SIZE = 128
HIDDEN = 128
BATCH = 16384
SEQ = 20
EPS = 1e-12

B_TOKENS = BATCH * SEQ
NUM_CORES = 2
NUM_SUBCORES = 16
NW = NUM_CORES * NUM_SUBCORES
B_PER_W = B_TOKENS // NW
CHUNK = 128
N_CHUNKS = B_PER_W // CHUNK


def _emb_sum_sc(wl_ids, pos_ids, hop_ids, wl_table, pos_table, hop_table):
    mesh = plsc.VectorSubcoreMesh(core_axis_name="c", subcore_axis_name="s")

    @functools.partial(
        pl.kernel,
        mesh=mesh,
        out_type=jax.ShapeDtypeStruct((B_TOKENS, HIDDEN), jnp.float32),
        scratch_types=[
            pltpu.VMEM((CHUNK,), jnp.int32),
            pltpu.VMEM((CHUNK,), jnp.int32),
            pltpu.VMEM((CHUNK,), jnp.int32),
            pltpu.VMEM((CHUNK, HIDDEN), jnp.float32),
            pltpu.VMEM((CHUNK, HIDDEN), jnp.float32),
            pltpu.VMEM((CHUNK, HIDDEN), jnp.float32),
            pltpu.SemaphoreType.DMA,
        ],
    )
    def k(wl_ids_h, pos_ids_h, hop_ids_h, wl_t, pos_t, hop_t, out_h,
          idx_a, idx_b, idx_c, buf_a, buf_b, buf_c, sem):
        wid = lax.axis_index("s") * NUM_CORES + lax.axis_index("c")

        def chunk_body(i, carry):
            base = wid * B_PER_W + i * CHUNK
            pltpu.sync_copy(wl_ids_h.at[pl.ds(base, CHUNK)], idx_a)
            pltpu.sync_copy(pos_ids_h.at[pl.ds(base, CHUNK)], idx_b)
            pltpu.sync_copy(hop_ids_h.at[pl.ds(base, CHUNK)], idx_c)
            ca = pltpu.async_copy(wl_t.at[idx_a], buf_a, sem)
            cb = pltpu.async_copy(pos_t.at[idx_b], buf_b, sem)
            cc = pltpu.async_copy(hop_t.at[idx_c], buf_c, sem)
            ca.wait()
            cb.wait()
            cc.wait()

            def row_body(r, c2):
                for g in range(HIDDEN // 16):
                    sl = pl.ds(g * 16, 16)
                    buf_a[r, sl] = buf_a[r, sl] + buf_b[r, sl] + buf_c[r, sl]
                return c2

            lax.fori_loop(0, CHUNK, row_body, 0)
            pltpu.sync_copy(buf_a, out_h.at[pl.ds(base, CHUNK)])
            return carry

        lax.fori_loop(0, N_CHUNKS, chunk_body, 0)

    return k(wl_ids, pos_ids, hop_ids, wl_table, pos_table, hop_table)


def _fused_proj_ln_body(raw_ref, emb_ref, w_ref, b_ref, g_ref, beta_ref, out_ref):
    x = jnp.dot(raw_ref[...], w_ref[...],
                preferred_element_type=jnp.float32,
                precision=lax.Precision.HIGHEST)
    x = x + b_ref[...] + emb_ref[...]
    mean = jnp.mean(x, axis=1, keepdims=True)
    xc = x - mean
    var = jnp.mean(xc * xc, axis=1, keepdims=True)
    xhat = xc * lax.rsqrt(var + EPS)
    out_ref[...] = xhat * g_ref[...] + beta_ref[...]


def _proj_add_ln_tc(raw2d, emb, W, b, gamma, beta):
    R = 2048
    grid = (B_TOKENS // R,)
    return pl.pallas_call(
        _fused_proj_ln_body,
        grid=grid,
        in_specs=[
            pl.BlockSpec((R, X_SIZE), lambda i: (i, 0)),
            pl.BlockSpec((R, HIDDEN), lambda i: (i, 0)),
            pl.BlockSpec((X_SIZE, HIDDEN), lambda i: (0, 0)),
            pl.BlockSpec((1, HIDDEN), lambda i: (0, 0)),
            pl.BlockSpec((1, HIDDEN), lambda i: (0, 0)),
            pl.BlockSpec((1, HIDDEN), lambda i: (0, 0)),
        ],
        out_specs=pl.BlockSpec((R, HIDDEN), lambda i: (i, 0)),
        out_shape=jax.ShapeDtypeStruct((B_TOKENS, HIDDEN), jnp.float32),
    )(raw2d, emb, W, b, gamma, beta)


def kernel(raw_features, wl_role_ids, init_pos_ids, hop_dis_ids, W, b,
           wl_table, pos_table, hop_table, gamma, beta):
    wl_f = wl_role_ids.astype(jnp.int32).reshape(-1)
    pos_f = init_pos_ids.astype(jnp.int32).reshape(-1)
    hop_f = hop_dis_ids.astype(jnp.int32).reshape(-1)
    emb = _emb_sum_sc(wl_f, pos_f, hop_f, wl_table, pos_table, hop_table)
    raw2d = raw_features.reshape(B_TOKENS, X_SIZE)
    out = _proj_add_ln_tc(raw2d, emb,
                          W, b.reshape(1, HIDDEN),
                          gamma.reshape(1, HIDDEN), beta.reshape(1, HIDDEN))
    return out.reshape(BATCH, SEQ, HIDDEN)
